# baseline (speedup 1.0000x reference)
import jax
import jax.numpy as jnp
from jax import lax
from jax.experimental import pallas as pl
from jax.experimental.pallas import tpu as pltpu
from jax.experimental.pallas import tpu_sc as plsc

_B, _T, _D, _E, _K = 1, 2048, 768, 8, 2
_NA = _T * _K
_MT = 128
_NPAD = _NA + _E * _MT
_NTILES = _NPAD // _MT
_NW = 32
_GP = _NPAD // _NW
_CH = _NA // 16
_TPW = _T // _NW
_L = 16


def _gate_body(x_ref, wg_ref, idx_ref, w_ref):
    x = x_ref[...]
    logits = jnp.dot(x, wg_ref[...], preferred_element_type=jnp.float32)
    lane = lax.broadcasted_iota(jnp.int32, logits.shape, 1)
    neg = jnp.float32(-jnp.inf)
    logits = jnp.where(lane < _E, logits, neg)
    m1 = jnp.max(logits, axis=1, keepdims=True)
    i1 = jnp.min(jnp.where(logits == m1, lane, _E), axis=1, keepdims=True)
    l2 = jnp.where(lane == i1, neg, logits)
    m2 = jnp.max(l2, axis=1, keepdims=True)
    i2 = jnp.min(jnp.where(l2 == m2, lane, _E), axis=1, keepdims=True)
    w1 = 1.0 / (1.0 + jnp.exp(m2 - m1))
    w2 = 1.0 - w1
    idx_ref[...] = jnp.where(lane == 0, i1, jnp.where(lane == 1, i2, 0))
    w_ref[...] = jnp.where(lane == 0, w1, jnp.where(lane == 1, w2, 0.0))


def _gate(x2, wgp, interpret=False):
    return pl.pallas_call(
        _gate_body,
        grid=(4,),
        in_specs=[
            pl.BlockSpec((_T // 4, _D), lambda i: (i, 0)),
            pl.BlockSpec((_D, 128), lambda i: (0, 0)),
        ],
        out_specs=[
            pl.BlockSpec((_T // 4, 128), lambda i: (i, 0)),
            pl.BlockSpec((_T // 4, 128), lambda i: (i, 0)),
        ],
        out_shape=[
            jax.ShapeDtypeStruct((_T, 128), jnp.int32),
            jax.ShapeDtypeStruct((_T, 128), jnp.float32),
        ],
        interpret=interpret,
    )(x2, wgp)


def _route_body(x_hbm, ef_hbm, wf_hbm, xs_hbm, slots_hbm, te_hbm, ws_hbm,
                efall_v, wv, slotidx_v, te_v, xrow_v, xrow2_v,
                sem, sem2, sem3, sem4, sem5, sem6):
    cid = lax.axis_index("c")
    sid = lax.axis_index("s")
    lanes = lax.iota(jnp.int32, _L)

    a0 = cid * _T + sid * 128

    lde = pltpu.async_copy(ef_hbm, efall_v, sem)
    ldw = [pltpu.async_copy(wf_hbm.at[pl.ds(a0 + q * 64, 64)], wv.at[q], sem2)
           for q in range(2)]
    lde.wait()

    def hist_body(v, carry):
        carry = list(carry)
        for u in range(4):
            evv = efall_v[pl.ds((v * 4 + u) * _L, _L)]
            before = lax.broadcast((v * 4 + u) * _L < a0, (_L,))
            for e in range(_E):
                m = jnp.where(evv == e, 1, 0)
                carry[e] = carry[e] + m
                carry[_E + e] = carry[_E + e] + jnp.where(before, m, 0)
        return tuple(carry)

    z = tuple(jnp.zeros((_L,), jnp.int32) for _ in range(2 * _E))
    acc = lax.fori_loop(0, _NA // _L // 4, hist_body, z)
    tot = [jnp.sum(acc[e]) for e in range(_E)]
    pre = [jnp.sum(acc[_E + e]) for e in range(_E)]
    g = [jnp.int32(0)]
    for e in range(1, _E):
        g.append((g[e - 1] + tot[e - 1] + (_MT - 1)) // _MT * _MT)

    @pl.when(jnp.logical_and(cid == 0, sid == 0))
    def _():
        for v in range(3):
            j = (lax.iota(jnp.int32, _L) + v * _L) * _MT
            acc2 = jnp.full((_L,), -1, jnp.int32)
            for e in range(_E):
                acc2 = acc2 + jnp.where(j >= lax.broadcast(g[e], (_L,)), 1, 0)
            te_v[pl.ds(v * _L, _L)] = jnp.minimum(acc2, _E - 1)
        pltpu.sync_copy(te_v, te_hbm)

    running = [lax.broadcast(g[e] + pre[e], (_L,)) for e in range(_E)]
    for v in range(8):
        evv = efall_v[pl.ds(a0 + v * _L, _L)]
        slot = jnp.zeros((_L,), jnp.int32)
        for e in range(_E):
            m = evv == e
            cs = plsc.cumsum(jnp.where(m, 1, 0))
            slot = jnp.where(m, running[e] + cs - 1, slot)
            running[e] = running[e] + lax.broadcast(cs[_L - 1], (_L,))
        slotidx_v[v // 4, pl.ds((v % 4) * _L, _L)] = slot

    for cp in ldw:
        cp.wait()
    pub = [pltpu.async_copy(slotidx_v.at[q],
                            slots_hbm.at[pl.ds(a0 + q * 64, 64)], sem3)
           for q in range(2)]
    wsc = [pltpu.async_copy(wv.at[q], ws_hbm.at[slotidx_v.at[q]], sem4)
           for q in range(2)]

    tb = sid * 128
    g0 = pltpu.async_copy(x_hbm.at[pl.ds(tb, 64)], xrow_v, sem)
    g1 = pltpu.async_copy(x_hbm.at[pl.ds(tb + 64, 64)], xrow2_v, sem2)
    g0.wait()
    s0 = pltpu.async_copy(xrow_v, xs_hbm.at[slotidx_v.at[0]], sem5)
    g1.wait()
    s1 = pltpu.async_copy(xrow2_v, xs_hbm.at[slotidx_v.at[1]], sem6)
    for cp in pub + wsc + [s0, s1]:
        cp.wait()


def _route(x2, ef, wf, interpret=False):
    mesh = plsc.VectorSubcoreMesh(core_axis_name="c", subcore_axis_name="s",
                                  num_cores=2, num_subcores=16)
    kern = pl.kernel(
        _route_body,
        out_type=[
            jax.ShapeDtypeStruct((_NPAD, _D), jnp.float32),
            jax.ShapeDtypeStruct((_NA,), jnp.int32),
            jax.ShapeDtypeStruct((48,), jnp.int32),
            jax.ShapeDtypeStruct((_NPAD,), jnp.float32),
        ],
        mesh=mesh,
        scratch_types=[
            pltpu.VMEM((_NA,), jnp.int32),
            pltpu.VMEM((2, 64), jnp.float32),
            pltpu.VMEM((2, 64), jnp.int32),
            pltpu.VMEM((48,), jnp.int32),
            pltpu.VMEM((64, _D), jnp.float32),
            pltpu.VMEM((64, _D), jnp.float32),
            pltpu.SemaphoreType.DMA,
            pltpu.SemaphoreType.DMA,
            pltpu.SemaphoreType.DMA,
            pltpu.SemaphoreType.DMA,
            pltpu.SemaphoreType.DMA,
            pltpu.SemaphoreType.DMA,
        ],
        compiler_params=pltpu.CompilerParams(needs_layout_passes=False),
        interpret=interpret,
    )
    return kern(x2, ef, wf)


def _gmm_body(te_ref, xs_ref, we_ref, ws_ref, y_ref):
    del te_ref
    y = jnp.dot(xs_ref[...].astype(jnp.bfloat16), we_ref[0],
                preferred_element_type=jnp.float32)
    y_ref[...] = y * ws_ref[...]


def _gmm(te, xs, web, ws, interpret=False):
    grid_spec = pltpu.PrefetchScalarGridSpec(
        num_scalar_prefetch=1,
        grid=(_NTILES,),
        in_specs=[
            pl.BlockSpec((_MT, _D), lambda i, te: (i, 0)),
            pl.BlockSpec((1, _D, _D), lambda i, te: (te[i], 0, 0)),
            pl.BlockSpec((_MT, 1), lambda i, te: (i, 0)),
        ],
        out_specs=pl.BlockSpec((_MT, _D), lambda i, te: (i, 0)),
    )
    return pl.pallas_call(
        _gmm_body,
        grid_spec=grid_spec,
        out_shape=jax.ShapeDtypeStruct((_NPAD, _D), jnp.float32),
        interpret=interpret,
    )(te, xs, web, ws)


def _combine_body(slots_hbm, y_hbm, out_hbm, sidx_v, sstage_v, r0_v, r1_v,
                  out_v, sems, osem):
    cid = lax.axis_index("c")
    sid = lax.axis_index("s")
    wid = cid * 16 + sid

    for c in range(2):
        tb = wid * _TPW + c * 32
        l0 = pltpu.async_copy(slots_hbm.at[pl.ds(tb, 32)],
                              sstage_v.at[0], sems[0])
        l1 = pltpu.async_copy(slots_hbm.at[pl.ds(_T + tb, 32)],
                              sstage_v.at[1], sems[1])
        l0.wait()
        l1.wait()
        for r in range(2):
            for i in range(2):
                sidx_v[r, pl.ds(i * _L, _L)] = jnp.minimum(
                    sstage_v[r, pl.ds(i * _L, _L)], _NPAD - 1)
        g0 = pltpu.async_copy(y_hbm.at[sidx_v.at[0]], r0_v, sems[2])
        g1 = pltpu.async_copy(y_hbm.at[sidx_v.at[1]], r1_v, sems[3])
        g0.wait()
        g1.wait()

        def body(i, _):
            for v in range(_D // _L):
                sl = pl.ds(v * _L, _L)
                out_v[i, sl] = r0_v[i, sl] + r1_v[i, sl]
            return 0

        lax.fori_loop(0, 32, body, 0)
        pltpu.sync_copy(out_v, out_hbm.at[pl.ds(tb, 32)])


def _combine(slots, y, interpret=False):
    mesh = plsc.VectorSubcoreMesh(core_axis_name="c", subcore_axis_name="s",
                                  num_cores=2, num_subcores=16)
    kern = pl.kernel(
        _combine_body,
        out_type=jax.ShapeDtypeStruct((_T, _D), jnp.float32),
        mesh=mesh,
        scratch_types=[
            pltpu.VMEM((2, 32), jnp.int32),
            pltpu.VMEM((2, 32), jnp.int32),
            pltpu.VMEM((32, _D), jnp.float32),
            pltpu.VMEM((32, _D), jnp.float32),
            pltpu.VMEM((32, _D), jnp.float32),
            [pltpu.SemaphoreType.DMA] * 4,
            pltpu.SemaphoreType.DMA,
        ],
        compiler_params=pltpu.CompilerParams(needs_layout_passes=False),
        interpret=interpret,
    )
    return kern(slots, y)


@jax.jit
def _pipeline(x2, wgp, web):
    idxp, wp = _gate(x2, wgp)
    ef = idxp[:, :_K].T.reshape(_NA)
    wf = wp[:, :_K].T.reshape(_NA)
    xs, slots, te, ws = _route(x2, ef, wf)
    y = _gmm(te, xs, web, ws.reshape(_NPAD, 1))
    out = _combine(slots, y)
    return out


def kernel(x, Wg, We):
    x2 = x.reshape(_T, _D)
    wgp = jnp.zeros((_D, 128), jnp.float32).at[:, :_E].set(Wg)
    web = We.astype(jnp.bfloat16)
    return _pipeline(x2, wgp, web).reshape(_B, _T, _D)


# speedup vs baseline: 1.0069x; 1.0069x over previous
import jax
import jax.numpy as jnp
from jax import lax
from jax.experimental import pallas as pl
from jax.experimental.pallas import tpu as pltpu
from jax.experimental.pallas import tpu_sc as plsc

_B, _T, _D, _E, _K = 1, 2048, 768, 8, 2
_NA = _T * _K
_MT = 128
_NPAD = _NA + _E * _MT
_NTILES = _NPAD // _MT
_NW = 32
_GP = _NPAD // _NW
_CH = _NA // 16
_TPW = _T // _NW
_L = 16


def _gate_body(x_ref, wg_ref, idx_ref, w_ref):
    x = x_ref[...]
    logits = jnp.dot(x, wg_ref[...], preferred_element_type=jnp.float32)
    lane = lax.broadcasted_iota(jnp.int32, logits.shape, 1)
    neg = jnp.float32(-jnp.inf)
    logits = jnp.where(lane < _E, logits, neg)
    m1 = jnp.max(logits, axis=1, keepdims=True)
    i1 = jnp.min(jnp.where(logits == m1, lane, _E), axis=1, keepdims=True)
    l2 = jnp.where(lane == i1, neg, logits)
    m2 = jnp.max(l2, axis=1, keepdims=True)
    i2 = jnp.min(jnp.where(l2 == m2, lane, _E), axis=1, keepdims=True)
    w1 = 1.0 / (1.0 + jnp.exp(m2 - m1))
    w2 = 1.0 - w1
    idx_ref[...] = jnp.where(lane == 0, i1, jnp.where(lane == 1, i2, 0))
    w_ref[...] = jnp.where(lane == 0, w1, jnp.where(lane == 1, w2, 0.0))


def _gate(x2, wgp, interpret=False):
    return pl.pallas_call(
        _gate_body,
        grid=(4,),
        in_specs=[
            pl.BlockSpec((_T // 4, _D), lambda i: (i, 0)),
            pl.BlockSpec((_D, 128), lambda i: (0, 0)),
        ],
        out_specs=[
            pl.BlockSpec((_T // 4, 128), lambda i: (i, 0)),
            pl.BlockSpec((_T // 4, 128), lambda i: (i, 0)),
        ],
        out_shape=[
            jax.ShapeDtypeStruct((_T, 128), jnp.int32),
            jax.ShapeDtypeStruct((_T, 128), jnp.float32),
        ],
        interpret=interpret,
    )(x2, wgp)


def _route_body(x_hbm, ef_hbm, wf_hbm, xs_hbm, slots_hbm, te_hbm, ws_hbm,
                efall_v, wv, slotidx_v, te_v, xrow_v, xrow2_v,
                sem, sem2, sem3, sem4, sem5, sem6):
    cid = lax.axis_index("c")
    sid = lax.axis_index("s")
    lanes = lax.iota(jnp.int32, _L)

    a0 = cid * _T + sid * 128

    lde = pltpu.async_copy(ef_hbm, efall_v, sem)
    ldw = [pltpu.async_copy(wf_hbm.at[pl.ds(a0 + q * 64, 64)], wv.at[q], sem2)
           for q in range(2)]
    lde.wait()

    def hist_body(v, carry):
        evv = efall_v[pl.ds(v * _L, _L)]
        before = lax.broadcast(v * _L < a0, (_L,))
        new = []
        for e in range(_E):
            m = jnp.where(evv == e, 1, 0)
            new.append(carry[e] + m)
            new.append(carry[_E + e] + jnp.where(before, m, 0))
        return tuple(new[0::2] + new[1::2])

    z = tuple(jnp.zeros((_L,), jnp.int32) for _ in range(2 * _E))
    acc = lax.fori_loop(0, _NA // _L, hist_body, z)
    tot = [jnp.sum(acc[e]) for e in range(_E)]
    pre = [jnp.sum(acc[_E + e]) for e in range(_E)]
    g = [jnp.int32(0)]
    for e in range(1, _E):
        g.append((g[e - 1] + tot[e - 1] + (_MT - 1)) // _MT * _MT)

    @pl.when(jnp.logical_and(cid == 0, sid == 0))
    def _():
        for v in range(3):
            j = (lax.iota(jnp.int32, _L) + v * _L) * _MT
            acc2 = jnp.full((_L,), -1, jnp.int32)
            for e in range(_E):
                acc2 = acc2 + jnp.where(j >= lax.broadcast(g[e], (_L,)), 1, 0)
            te_v[pl.ds(v * _L, _L)] = jnp.minimum(acc2, _E - 1)
        pltpu.sync_copy(te_v, te_hbm)

    running = [lax.broadcast(g[e] + pre[e], (_L,)) for e in range(_E)]
    for v in range(8):
        evv = efall_v[pl.ds(a0 + v * _L, _L)]
        slot = jnp.zeros((_L,), jnp.int32)
        for e in range(_E):
            m = evv == e
            cs = plsc.cumsum(jnp.where(m, 1, 0))
            slot = jnp.where(m, running[e] + cs - 1, slot)
            running[e] = running[e] + lax.broadcast(cs[_L - 1], (_L,))
        slotidx_v[v // 4, pl.ds((v % 4) * _L, _L)] = slot

    for cp in ldw:
        cp.wait()
    pub = [pltpu.async_copy(slotidx_v.at[q],
                            slots_hbm.at[pl.ds(a0 + q * 64, 64)], sem3)
           for q in range(2)]
    wsc = [pltpu.async_copy(wv.at[q], ws_hbm.at[slotidx_v.at[q]], sem4)
           for q in range(2)]

    tb = sid * 128
    g0 = pltpu.async_copy(x_hbm.at[pl.ds(tb, 64)], xrow_v, sem)
    g1 = pltpu.async_copy(x_hbm.at[pl.ds(tb + 64, 64)], xrow2_v, sem2)
    g0.wait()
    s0 = pltpu.async_copy(xrow_v, xs_hbm.at[slotidx_v.at[0]], sem5)
    g1.wait()
    s1 = pltpu.async_copy(xrow2_v, xs_hbm.at[slotidx_v.at[1]], sem6)
    for cp in pub + wsc + [s0, s1]:
        cp.wait()


def _route(x2, ef, wf, interpret=False):
    mesh = plsc.VectorSubcoreMesh(core_axis_name="c", subcore_axis_name="s",
                                  num_cores=2, num_subcores=16)
    kern = pl.kernel(
        _route_body,
        out_type=[
            jax.ShapeDtypeStruct((_NPAD, _D), jnp.float32),
            jax.ShapeDtypeStruct((_NA,), jnp.int32),
            jax.ShapeDtypeStruct((48,), jnp.int32),
            jax.ShapeDtypeStruct((_NPAD,), jnp.float32),
        ],
        mesh=mesh,
        scratch_types=[
            pltpu.VMEM((_NA,), jnp.int32),
            pltpu.VMEM((2, 64), jnp.float32),
            pltpu.VMEM((2, 64), jnp.int32),
            pltpu.VMEM((48,), jnp.int32),
            pltpu.VMEM((64, _D), jnp.float32),
            pltpu.VMEM((64, _D), jnp.float32),
            pltpu.SemaphoreType.DMA,
            pltpu.SemaphoreType.DMA,
            pltpu.SemaphoreType.DMA,
            pltpu.SemaphoreType.DMA,
            pltpu.SemaphoreType.DMA,
            pltpu.SemaphoreType.DMA,
        ],
        compiler_params=pltpu.CompilerParams(needs_layout_passes=False),
        interpret=interpret,
    )
    return kern(x2, ef, wf)


def _gmm_body(te_ref, xs_ref, we_ref, ws_ref, y_ref):
    del te_ref
    y = jnp.dot(xs_ref[...].astype(jnp.bfloat16), we_ref[0],
                preferred_element_type=jnp.float32)
    y_ref[...] = y * ws_ref[...]


def _gmm(te, xs, web, ws, interpret=False):
    grid_spec = pltpu.PrefetchScalarGridSpec(
        num_scalar_prefetch=1,
        grid=(_NTILES,),
        in_specs=[
            pl.BlockSpec((_MT, _D), lambda i, te: (i, 0)),
            pl.BlockSpec((1, _D, _D), lambda i, te: (te[i], 0, 0)),
            pl.BlockSpec((_MT, 1), lambda i, te: (i, 0)),
        ],
        out_specs=pl.BlockSpec((_MT, _D), lambda i, te: (i, 0)),
    )
    return pl.pallas_call(
        _gmm_body,
        grid_spec=grid_spec,
        out_shape=jax.ShapeDtypeStruct((_NPAD, _D), jnp.float32),
        interpret=interpret,
    )(te, xs, web, ws)


def _combine_body(slots_hbm, y_hbm, out_hbm, sidx_v, sstage_v, r0_v, r1_v,
                  out_v, sems, osem):
    cid = lax.axis_index("c")
    sid = lax.axis_index("s")
    wid = cid * 16 + sid

    for c in range(2):
        tb = wid * _TPW + c * 32
        l0 = pltpu.async_copy(slots_hbm.at[pl.ds(tb, 32)],
                              sstage_v.at[0], sems[0])
        l1 = pltpu.async_copy(slots_hbm.at[pl.ds(_T + tb, 32)],
                              sstage_v.at[1], sems[1])
        l0.wait()
        l1.wait()
        for r in range(2):
            for i in range(2):
                sidx_v[r, pl.ds(i * _L, _L)] = jnp.minimum(
                    sstage_v[r, pl.ds(i * _L, _L)], _NPAD - 1)
        g0 = pltpu.async_copy(y_hbm.at[sidx_v.at[0]], r0_v, sems[2])
        g1 = pltpu.async_copy(y_hbm.at[sidx_v.at[1]], r1_v, sems[3])
        g0.wait()
        g1.wait()

        def body(i, _):
            for v in range(_D // _L):
                sl = pl.ds(v * _L, _L)
                out_v[i, sl] = r0_v[i, sl] + r1_v[i, sl]
            return 0

        lax.fori_loop(0, 32, body, 0)
        pltpu.sync_copy(out_v, out_hbm.at[pl.ds(tb, 32)])


def _combine(slots, y, interpret=False):
    mesh = plsc.VectorSubcoreMesh(core_axis_name="c", subcore_axis_name="s",
                                  num_cores=2, num_subcores=16)
    kern = pl.kernel(
        _combine_body,
        out_type=jax.ShapeDtypeStruct((_T, _D), jnp.float32),
        mesh=mesh,
        scratch_types=[
            pltpu.VMEM((2, 32), jnp.int32),
            pltpu.VMEM((2, 32), jnp.int32),
            pltpu.VMEM((32, _D), jnp.float32),
            pltpu.VMEM((32, _D), jnp.float32),
            pltpu.VMEM((32, _D), jnp.float32),
            [pltpu.SemaphoreType.DMA] * 4,
            pltpu.SemaphoreType.DMA,
        ],
        compiler_params=pltpu.CompilerParams(needs_layout_passes=False),
        interpret=interpret,
    )
    return kern(slots, y)


@jax.jit
def _pipeline(x2, wgp, web):
    idxp, wp = _gate(x2, wgp)
    ef = idxp[:, :_K].T.reshape(_NA)
    wf = wp[:, :_K].T.reshape(_NA)
    xs, slots, te, ws = _route(x2, ef, wf)
    y = _gmm(te, xs, web, ws.reshape(_NPAD, 1))
    out = _combine(slots, y)
    return out


def kernel(x, Wg, We):
    x2 = x.reshape(_T, _D)
    wgp = jnp.zeros((_D, 128), jnp.float32).at[:, :_E].set(Wg)
    web = We.astype(jnp.bfloat16)
    return _pipeline(x2, wgp, web).reshape(_B, _T, _D)
